# baseline (speedup 1.0000x reference)
import functools

import jax
import jax.numpy as jnp
from jax import lax
from jax.experimental import pallas as pl
from jax.experimental.pallas import tpu as pltpu
from jax.experimental.pallas import tpu_sc as plsc

NC, NS, L = 2, 16, 16
NW = NC * NS
BATCH = 16384
NF = 100
NCONT = 20
NCAT = 80
D = 16
NB = 64
ROWS_PER_W = BATCH // NW
NCHUNK = ROWS_PER_W // NB


def kernel(inputs, categorical_embeddings, continuous_embeddings, bias):
    mesh = plsc.VectorSubcoreMesh(
        core_axis_name="c", subcore_axis_name="s", num_cores=NC, num_subcores=NS
    )

    @functools.partial(
        pl.kernel,
        out_type=jax.ShapeDtypeStruct((BATCH * NF, D), jnp.float32),
        mesh=mesh,
        compiler_params=pltpu.CompilerParams(
            needs_layout_passes=False, use_tc_tiling_on_sc=False
        ),
        scratch_types=[
            pltpu.VMEM((NB, NF), jnp.float32),
            pltpu.VMEM((NB, NCAT), jnp.int32),
            pltpu.VMEM((NB * NF, D), jnp.float32),
            pltpu.VMEM((NF, D), jnp.float32),
            pltpu.VMEM((NCONT, D), jnp.float32),
            pltpu.VMEM((NF,), jnp.float32),
            pltpu.SemaphoreType.DMA,
        ],
    )
    def sc_kernel(in_hbm, table_hbm, cont_hbm, bias_hbm, out_hbm,
                  chunk, idx, obuf, bb, ce, bv, gsem):
        wid = lax.axis_index("s") * NC + lax.axis_index("c")
        base0 = wid * ROWS_PER_W

        pltpu.sync_copy(cont_hbm, ce)
        pltpu.sync_copy(bias_hbm, bv)
        for f in range(NF):
            bb[f, :] = plsc.load_gather(bv, [jnp.full((L,), f, jnp.int32)])

        def chunk_body(c, carry):
            base = base0 + c * NB
            pltpu.sync_copy(in_hbm.at[pl.ds(base, NB)], chunk)

            def idx_body(i, _):
                for k in range(NCAT // L):
                    xv = chunk[i, pl.ds(NCONT + L * k, L)]
                    offs = 1 + (L * k + lax.iota(jnp.int32, L)) * 10000
                    idx[i, pl.ds(L * k, L)] = xv.astype(jnp.int32) + offs
                return _
            lax.fori_loop(0, NB, idx_body, 0)

            def g_issue(i, _):
                pltpu.async_copy(
                    table_hbm.at[idx.at[i]], obuf.at[pl.ds(i * NF + NCONT, NCAT)],
                    gsem,
                )
                return _
            lax.fori_loop(0, NB, g_issue, 0)

            def cont_body(i, _):
                iv = jnp.full((L,), i, jnp.int32)
                for j in range(NCONT):
                    xb = plsc.load_gather(chunk, [iv, jnp.full((L,), j, jnp.int32)])
                    obuf[i * NF + j, :] = xb * ce[j, :] + bb[j, :]
                return _
            lax.fori_loop(0, NB, cont_body, 0)

            def g_drain(i, _):
                pltpu.make_async_copy(
                    table_hbm.at[idx.at[i]], obuf.at[pl.ds(i * NF + NCONT, NCAT)],
                    gsem,
                ).wait()
                return _
            lax.fori_loop(0, NB, g_drain, 0)

            def bias_body(f, _):
                bvec = bb[NCONT + f, :]
                def row_body(i, __):
                    r = i * NF + NCONT + f
                    obuf[r, :] = obuf[r, :] + bvec
                    return __
                lax.fori_loop(0, NB, row_body, 0)
                return _
            lax.fori_loop(0, NCAT, bias_body, 0)

            pltpu.sync_copy(obuf, out_hbm.at[pl.ds(base * NF, NB * NF)])
            return carry

        lax.fori_loop(0, NCHUNK, chunk_body, 0)

    out = sc_kernel(inputs, categorical_embeddings, continuous_embeddings, bias)
    return out.reshape(BATCH, NF, D)


# speedup vs baseline: 1.0313x; 1.0313x over previous
import functools

import jax
import jax.numpy as jnp
from jax import lax
from jax.experimental import pallas as pl
from jax.experimental.pallas import tpu as pltpu
from jax.experimental.pallas import tpu_sc as plsc

NC, NS, L = 2, 16, 16
NW = NC * NS
BATCH = 16384
NF = 100
NCONT = 20
NCAT = 80
D = 16
NB = 64
ROWS_PER_W = BATCH // NW
NCHUNK = ROWS_PER_W // NB


def kernel(inputs, categorical_embeddings, continuous_embeddings, bias):
    mesh = plsc.VectorSubcoreMesh(
        core_axis_name="c", subcore_axis_name="s", num_cores=NC, num_subcores=NS
    )

    @functools.partial(
        pl.kernel,
        out_type=jax.ShapeDtypeStruct((BATCH * NF, D), jnp.float32),
        mesh=mesh,
        compiler_params=pltpu.CompilerParams(
            needs_layout_passes=False, use_tc_tiling_on_sc=False
        ),
        scratch_types=[
            pltpu.VMEM((NB, NF), jnp.float32),
            pltpu.VMEM((NB, NCAT), jnp.int32),
            pltpu.VMEM((NB * NF, D), jnp.float32),
            pltpu.VMEM((NF, D), jnp.float32),
            pltpu.VMEM((NCONT, D), jnp.float32),
            pltpu.VMEM((NF,), jnp.float32),
            pltpu.SemaphoreType.DMA,
        ],
    )
    def sc_kernel(in_hbm, table_hbm, cont_hbm, bias_hbm, out_hbm,
                  chunk, idx, obuf, bb, ce, bv, gsem):
        wid = lax.axis_index("s") * NC + lax.axis_index("c")
        base0 = wid * ROWS_PER_W

        pltpu.sync_copy(cont_hbm, ce)
        pltpu.sync_copy(bias_hbm, bv)
        for f in range(NF):
            bb[f, :] = plsc.load_gather(bv, [jnp.full((L,), f, jnp.int32)])

        def chunk_body(c, carry):
            base = base0 + c * NB
            pltpu.sync_copy(in_hbm.at[pl.ds(base, NB)], chunk)

            def row_body(i, _):
                r0 = i * NF
                for f in range(NCAT):
                    obuf[r0 + NCONT + f, :] = bb[NCONT + f, :]
                for k in range(NCAT // L):
                    xv = chunk[i, pl.ds(NCONT + L * k, L)]
                    offs = 1 + (L * k + lax.iota(jnp.int32, L)) * 10000
                    idx[i, pl.ds(L * k, L)] = xv.astype(jnp.int32) + offs
                pltpu.async_copy(
                    table_hbm.at[idx.at[i]], obuf.at[pl.ds(r0 + NCONT, NCAT)],
                    gsem, add=True,
                )
                return _
            lax.fori_loop(0, NB, row_body, 0)

            def cont_body(i, _):
                iv = jnp.full((L,), i, jnp.int32)
                r0 = i * NF
                for j in range(NCONT):
                    xb = plsc.load_gather(chunk, [iv, jnp.full((L,), j, jnp.int32)])
                    obuf[r0 + j, :] = xb * ce[j, :] + bb[j, :]
                return _
            lax.fori_loop(0, NB, cont_body, 0)

            def g_drain(i, _):
                pltpu.make_async_copy(
                    table_hbm.at[idx.at[i]], obuf.at[pl.ds(i * NF + NCONT, NCAT)],
                    gsem,
                ).wait()
                return _
            lax.fori_loop(0, NB, g_drain, 0)

            pltpu.sync_copy(obuf, out_hbm.at[pl.ds(base * NF, NB * NF)])
            return carry

        lax.fori_loop(0, NCHUNK, chunk_body, 0)

    out = sc_kernel(inputs, categorical_embeddings, continuous_embeddings, bias)
    return out.reshape(BATCH, NF, D)


# speedup vs baseline: 3.6057x; 3.4962x over previous
import functools

import jax
import jax.numpy as jnp
from jax import lax
from jax.experimental import pallas as pl
from jax.experimental.pallas import tpu as pltpu
from jax.experimental.pallas import tpu_sc as plsc

NC, NS, L = 2, 16, 16
NW = NC * NS
BATCH = 16384
NF = 100
NCONT = 20
NCAT = 80
D = 16
BH = 128
NBH = BATCH // BH
TILES_PER_W = NBH // NW
FB = 20
NFB = NCAT // FB


def kernel(inputs, categorical_embeddings, continuous_embeddings, bias):
    mesh = plsc.VectorSubcoreMesh(
        core_axis_name="c", subcore_axis_name="s", num_cores=NC, num_subcores=NS
    )

    @functools.partial(
        pl.kernel,
        out_type=jax.ShapeDtypeStruct((NF * 2 * NBH * 8 * BH,), jnp.float32),
        mesh=mesh,
        compiler_params=pltpu.CompilerParams(
            needs_layout_passes=False, use_tc_tiling_on_sc=False
        ),
        scratch_types=[
            pltpu.VMEM((NF, BH), jnp.float32),
            pltpu.VMEM((NCAT, BH), jnp.int32),
            pltpu.VMEM((FB * BH, D), jnp.float32),
            pltpu.VMEM((FB * D * BH,), jnp.float32),
            pltpu.VMEM((NF, D), jnp.float32),
            pltpu.VMEM((NCONT * D, D), jnp.float32),
            pltpu.VMEM((FB, L), jnp.int32),
            pltpu.VMEM((L + NCONT * D,), jnp.float32),
            pltpu.VMEM((L + NF,), jnp.float32),
            pltpu.SemaphoreType.DMA,
            pltpu.SemaphoreType.DMA,
        ],
    )
    def sc_kernel(xt_hbm, table_hbm, cont_hbm, bias_hbm, out_hbm,
                  xchunk, idxb, gbuf, stag, bb, cesp, sbase, ce, bv,
                  gsem, osem):
        wid = lax.axis_index("s") * NC + lax.axis_index("c")

        pltpu.sync_copy(cont_hbm, ce.at[pl.ds(L, NCONT * D)])
        pltpu.sync_copy(bias_hbm, bv.at[pl.ds(L, NF)])
        for f in range(NF):
            bb[f, :] = plsc.load_gather(bv, [jnp.full((L,), L + f, jnp.int32)])
        for f in range(NCONT):
            for d in range(D):
                cesp[f * D + d, :] = plsc.load_gather(
                    ce, [jnp.full((L,), L + f * D + d, jnp.int32)]
                )
        for fl in range(FB):
            sbase[fl, :] = (fl * D + lax.iota(jnp.int32, L)) * BH

        def tile_body(c, carry):
            bh = wid * TILES_PER_W + c
            b0 = bh * BH
            pltpu.sync_copy(xt_hbm.at[:, pl.ds(b0, BH)], xchunk)

            def idx_body(f, _):
                offs = jnp.full((L,), 1, jnp.int32) + f * 10000
                for k in range(BH // L):
                    xv = xchunk[NCONT + f, pl.ds(L * k, L)]
                    idxb[f, pl.ds(L * k, L)] = xv.astype(jnp.int32) + offs
                return _
            lax.fori_loop(0, NCAT, idx_body, 0)

            def cont_body(f, _):
                biasv = bb[f, :]
                for k in range(BH // L):
                    xv = xchunk[f, pl.ds(L * k, L)]
                    for d in range(D):
                        ev = cesp[f * D + d, :]
                        stag[pl.ds((f * D + d) * BH + L * k, L)] = xv * ev + biasv
                return _
            lax.fori_loop(0, NCONT, cont_body, 0)
            for fl in range(FB):
                for dh in range(2):
                    r = ((fl * 2 + dh) * NBH + bh) * 8 * BH
                    pltpu.async_copy(
                        stag.at[pl.ds((fl * D + dh * 8) * BH, 8 * BH)],
                        out_hbm.at[pl.ds(r, 8 * BH)], osem,
                    )

            for fb in range(NFB):
                for fl in range(FB):
                    f = fb * FB + fl
                    pltpu.async_copy(
                        table_hbm.at[idxb.at[f]],
                        gbuf.at[pl.ds(fl * BH, BH)], gsem,
                    )
                for fl in range(FB):
                    f = fb * FB + fl
                    pltpu.make_async_copy(
                        table_hbm.at[idxb.at[f]],
                        gbuf.at[pl.ds(fl * BH, BH)], gsem,
                    ).wait()
                for _ in range(2 * FB):
                    pltpu.make_async_copy(
                        stag.at[pl.ds(0, 8 * BH)],
                        out_hbm.at[pl.ds(0, 8 * BH)], osem,
                    ).wait()

                def sc_body(b, _):
                    bvv = jnp.full((L,), b, jnp.int32)
                    for fl in range(FB):
                        row = gbuf[fl * BH + b, :] + bb[NCONT + fb * FB + fl, :]
                        plsc.store_scatter(stag, [sbase[fl, :] + bvv], row)
                    return _
                lax.fori_loop(0, BH, sc_body, 0)

                for fl in range(FB):
                    fo = NCONT + fb * FB + fl
                    for dh in range(2):
                        r = ((fo * 2 + dh) * NBH + bh) * 8 * BH
                        pltpu.async_copy(
                            stag.at[pl.ds((fl * D + dh * 8) * BH, 8 * BH)],
                            out_hbm.at[pl.ds(r, 8 * BH)], osem,
                        )
            for _ in range(2 * FB):
                pltpu.make_async_copy(
                    stag.at[pl.ds(0, 8 * BH)],
                    out_hbm.at[pl.ds(0, 8 * BH)], osem,
                ).wait()
            return carry

        lax.fori_loop(0, TILES_PER_W, tile_body, 0)

    out = sc_kernel(
        inputs.T, categorical_embeddings, continuous_embeddings.reshape(-1), bias
    )
    out = out.reshape(NF, 2, NBH, 8, BH).transpose(2, 4, 0, 1, 3)
    return out.reshape(BATCH, NF, D)


# speedup vs baseline: 3.8823x; 1.0767x over previous
import functools

import jax
import jax.numpy as jnp
from jax import lax
from jax.experimental import pallas as pl
from jax.experimental.pallas import tpu as pltpu
from jax.experimental.pallas import tpu_sc as plsc

NC, NS, L = 2, 16, 16
NW = NC * NS
BATCH = 16384
NF = 100
NCONT = 20
NCAT = 80
D = 16
BH = 128
NBH = BATCH // BH
TILES_PER_W = NBH // NW
FB = 10
NBLK = NF // FB


def kernel(inputs, categorical_embeddings, continuous_embeddings, bias):
    mesh = plsc.VectorSubcoreMesh(
        core_axis_name="c", subcore_axis_name="s", num_cores=NC, num_subcores=NS
    )

    @functools.partial(
        pl.kernel,
        out_type=jax.ShapeDtypeStruct((NF * 2 * NBH * 8 * BH,), jnp.float32),
        mesh=mesh,
        compiler_params=pltpu.CompilerParams(
            needs_layout_passes=False, use_tc_tiling_on_sc=False
        ),
        scratch_types=[
            pltpu.VMEM((NF, BH), jnp.float32),
            pltpu.VMEM((NCAT, BH), jnp.int32),
            pltpu.VMEM((2, FB * BH, D), jnp.float32),
            pltpu.VMEM((2, FB * D * BH), jnp.float32),
            pltpu.VMEM((NF, D), jnp.float32),
            pltpu.VMEM((NCONT * D, D), jnp.float32),
            pltpu.VMEM((FB, L), jnp.int32),
            pltpu.VMEM((L + NCONT * D,), jnp.float32),
            pltpu.VMEM((L + NF,), jnp.float32),
            pltpu.SemaphoreType.DMA,
            pltpu.SemaphoreType.DMA,
            pltpu.SemaphoreType.DMA,
        ],
    )
    def sc_kernel(xt_hbm, table_hbm, cont_hbm, bias_hbm, out_hbm,
                  xchunk, idxb, gbuf, stag, bb, cesp, sbase, ce, bv,
                  gsem, osem0, osem1):
        osems = (osem0, osem1)
        wid = lax.axis_index("s") * NC + lax.axis_index("c")

        pltpu.sync_copy(cont_hbm, ce.at[pl.ds(L, NCONT * D)])
        pltpu.sync_copy(bias_hbm, bv.at[pl.ds(L, NF)])
        for f in range(NF):
            bb[f, :] = plsc.load_gather(bv, [jnp.full((L,), L + f, jnp.int32)])
        for f in range(NCONT):
            for d in range(D):
                cesp[f * D + d, :] = plsc.load_gather(
                    ce, [jnp.full((L,), L + f * D + d, jnp.int32)]
                )
        for fl in range(FB):
            sbase[fl, :] = (fl * D + lax.iota(jnp.int32, L)) * BH

        def fire_gathers(blk):
            p = blk & 1
            for fl in range(FB):
                fc = blk * FB - NCONT + fl
                pltpu.async_copy(
                    table_hbm.at[idxb.at[fc]],
                    gbuf.at[p, pl.ds(fl * BH, BH)], gsem,
                )

        def wait_gathers(blk):
            p = blk & 1
            for fl in range(FB):
                fc = blk * FB - NCONT + fl
                pltpu.make_async_copy(
                    table_hbm.at[idxb.at[fc]],
                    gbuf.at[p, pl.ds(fl * BH, BH)], gsem,
                ).wait()

        def drain_out(p):
            for _ in range(2 * FB):
                pltpu.make_async_copy(
                    stag.at[0, pl.ds(0, 8 * BH)],
                    out_hbm.at[pl.ds(0, 8 * BH)], osems[p],
                ).wait()

        def fire_out(blk, bh):
            p = blk & 1
            for fl in range(FB):
                fo = blk * FB + fl
                for dh in range(2):
                    r = ((fo * 2 + dh) * NBH + bh) * 8 * BH
                    pltpu.async_copy(
                        stag.at[p, pl.ds((fl * D + dh * 8) * BH, 8 * BH)],
                        out_hbm.at[pl.ds(r, 8 * BH)], osems[p],
                    )

        def tile_body(c, carry):
            bh = wid * TILES_PER_W + c
            b0 = bh * BH
            pltpu.sync_copy(xt_hbm.at[:, pl.ds(b0, BH)], xchunk)

            def idx_body(f, _):
                offs = jnp.full((L,), 1, jnp.int32) + f * 10000
                for k in range(BH // L):
                    xv = xchunk[NCONT + f, pl.ds(L * k, L)]
                    idxb[f, pl.ds(L * k, L)] = xv.astype(jnp.int32) + offs
                return _
            lax.fori_loop(0, NCAT, idx_body, 0)

            fire_gathers(2)
            for blk in range(NBLK):
                p = blk & 1
                if 3 <= blk + 1 < NBLK:
                    fire_gathers(blk + 1)
                if blk >= 2:
                    wait_gathers(blk)
                if blk >= 2:
                    drain_out(p)
                if blk < 2:
                    def cont_body(fl, _):
                        f = blk * FB + fl
                        biasv = bb[f, :]
                        for k in range(BH // L):
                            xv = xchunk[f, pl.ds(L * k, L)]
                            for d in range(D):
                                ev = cesp[f * D + d, :]
                                stag[p, pl.ds((fl * D + d) * BH + L * k, L)] = (
                                    xv * ev + biasv
                                )
                        return _
                    lax.fori_loop(0, FB, cont_body, 0)
                else:
                    def sc_body(b, _):
                        bvv = jnp.full((L,), b, jnp.int32)
                        for fl in range(FB):
                            row = (
                                gbuf[p, fl * BH + b, :]
                                + bb[blk * FB + fl, :]
                            )
                            plsc.store_scatter(
                                stag.at[p], [sbase[fl, :] + bvv], row
                            )
                        return _
                    lax.fori_loop(0, BH, sc_body, 0)
                fire_out(blk, bh)
            drain_out(0)
            drain_out(1)
            return carry

        lax.fori_loop(0, TILES_PER_W, tile_body, 0)

    out = sc_kernel(
        inputs.T, categorical_embeddings, continuous_embeddings.reshape(-1), bias
    )
    out = out.reshape(NF, 2, NBH, 8, BH).transpose(2, 4, 0, 1, 3)
    return out.reshape(BATCH, NF, D)


# speedup vs baseline: 4.1582x; 1.0711x over previous
import functools

import jax
import jax.numpy as jnp
from jax import lax
from jax.experimental import pallas as pl
from jax.experimental.pallas import tpu as pltpu
from jax.experimental.pallas import tpu_sc as plsc

NC, NS, L = 2, 16, 16
NW = NC * NS
BATCH = 16384
NF = 100
NCONT = 20
NCAT = 80
D = 16
TOTAL_TOKENS = 800001
BH = 128
NBH = BATCH // BH
TILES_PER_W = NBH // NW
FB = 10
NBLK = NF // FB


def kernel(inputs, categorical_embeddings, continuous_embeddings, bias):
    mesh = plsc.VectorSubcoreMesh(
        core_axis_name="c", subcore_axis_name="s", num_cores=NC, num_subcores=NS
    )

    @functools.partial(
        pl.kernel,
        out_type=jax.ShapeDtypeStruct((NF * 2 * NBH * 8 * BH,), jnp.float32),
        mesh=mesh,
        compiler_params=pltpu.CompilerParams(
            needs_layout_passes=False, use_tc_tiling_on_sc=False
        ),
        scratch_types=[
            pltpu.VMEM((NF, BH), jnp.float32),
            pltpu.VMEM((NCAT, BH), jnp.int32),
            pltpu.VMEM((2, FB * BH, D), jnp.float32),
            pltpu.VMEM((2, FB * D * BH), jnp.float32),
            pltpu.VMEM((NF, D), jnp.float32),
            pltpu.VMEM((NCONT * D, D), jnp.float32),
            pltpu.VMEM((FB, L), jnp.int32),
            pltpu.VMEM((L + NCONT * D,), jnp.float32),
            pltpu.VMEM((L + NF,), jnp.float32),
            pltpu.SemaphoreType.DMA,
            pltpu.SemaphoreType.DMA,
            pltpu.SemaphoreType.DMA,
        ],
    )
    def sc_kernel(xt_hbm, table_hbm, cont_hbm, bias_hbm, out_hbm,
                  xchunk, idxb, gbuf, stag, bb, cesp, sbase, ce, bv,
                  gsem, osem0, osem1):
        osems = (osem0, osem1)
        wid = lax.axis_index("s") * NC + lax.axis_index("c")

        pltpu.sync_copy(cont_hbm, ce.at[pl.ds(L, NCONT * D)])
        pltpu.sync_copy(bias_hbm, bv.at[pl.ds(L, NF)])
        for f in range(NF):
            bb[f, :] = plsc.load_gather(bv, [jnp.full((L,), L + f, jnp.int32)])
        for f in range(NCONT):
            for d in range(D):
                cesp[f * D + d, :] = plsc.load_gather(
                    ce, [jnp.full((L,), L + f * D + d, jnp.int32)]
                )
        for fl in range(FB):
            sbase[fl, :] = (fl * D + lax.iota(jnp.int32, L)) * BH

        def fire_gathers(blk):
            p = blk & 1
            for fl in range(FB):
                fc = blk * FB - NCONT + fl
                pltpu.async_copy(
                    table_hbm.at[idxb.at[fc]],
                    gbuf.at[p, pl.ds(fl * BH, BH)], gsem,
                )

        def wait_gathers(blk):
            p = blk & 1
            for fl in range(FB):
                fc = blk * FB - NCONT + fl
                pltpu.make_async_copy(
                    table_hbm.at[idxb.at[fc]],
                    gbuf.at[p, pl.ds(fl * BH, BH)], gsem,
                ).wait()

        def drain_out(p):
            for _ in range(2 * FB):
                pltpu.make_async_copy(
                    stag.at[0, pl.ds(0, 8 * BH)],
                    out_hbm.at[pl.ds(0, 8 * BH)], osems[p],
                ).wait()

        def fire_out(blk, bh):
            p = blk & 1
            for fl in range(FB):
                fo = blk * FB + fl
                for dh in range(2):
                    r = ((fo * 2 + dh) * NBH + bh) * 8 * BH
                    pltpu.async_copy(
                        stag.at[p, pl.ds((fl * D + dh * 8) * BH, 8 * BH)],
                        out_hbm.at[pl.ds(r, 8 * BH)], osems[p],
                    )

        def tile_body(c, carry):
            bh = wid * TILES_PER_W + c
            b0 = bh * BH
            pltpu.sync_copy(xt_hbm.at[:, pl.ds(b0, BH)], xchunk)

            def idx_body(f, _):
                offs = jnp.full((L,), 1, jnp.int32) + f * 10000
                for k in range(BH // L):
                    xv = xchunk[NCONT + f, pl.ds(L * k, L)]
                    idxb[f, pl.ds(L * k, L)] = xv.astype(jnp.int32) + offs
                return _
            lax.fori_loop(0, NCAT, idx_body, 0)

            fire_gathers(2)
            for blk in range(NBLK):
                p = blk & 1
                if 3 <= blk + 1 < NBLK:
                    fire_gathers(blk + 1)
                if blk >= 2:
                    wait_gathers(blk)
                if blk >= 2:
                    drain_out(p)
                if blk < 2:
                    def cont_body(fl, _):
                        f = blk * FB + fl
                        biasv = bb[f, :]
                        for k in range(BH // L):
                            xv = xchunk[f, pl.ds(L * k, L)]
                            for d in range(D):
                                ev = cesp[f * D + d, :]
                                stag[p, pl.ds((fl * D + d) * BH + L * k, L)] = (
                                    xv * ev + biasv
                                )
                        return _
                    lax.fori_loop(0, FB, cont_body, 0)
                else:
                    biases = [bb[blk * FB + fl, :] for fl in range(FB)]
                    sbs = [sbase[fl, :] for fl in range(FB)]
                    def sc_body(b, _):
                        bvv = jnp.full((L,), b, jnp.int32)
                        for fl in range(FB):
                            row = gbuf[p, fl * BH + b, :] + biases[fl]
                            plsc.store_scatter(
                                stag.at[p], [sbs[fl] + bvv], row
                            )
                        return _
                    lax.fori_loop(0, BH, sc_body, 0)
                fire_out(blk, bh)
            drain_out(0)
            drain_out(1)
            return carry

        lax.fori_loop(0, TILES_PER_W, tile_body, 0)

    out = sc_kernel(
        inputs.T, categorical_embeddings, continuous_embeddings.reshape(-1), bias
    )
    out = out.reshape(NF, 2, NBH, 8, BH).transpose(2, 4, 0, 1, 3)
    return out.reshape(BATCH, NF, D)


# speedup vs baseline: 5.1707x; 1.2435x over previous
import functools

import jax
import jax.numpy as jnp
from jax import lax
from jax.experimental import pallas as pl
from jax.experimental.pallas import tpu as pltpu
from jax.experimental.pallas import tpu_sc as plsc

NC, NS, L = 2, 16, 16
NW = NC * NS
BATCH = 16384
NF = 100
NCONT = 20
NCAT = 80
D = 16
TOTAL_TOKENS = 800001
BH = 128
NBH = BATCH // BH
TILES_PER_W = NBH // NW
FB = 10
NBLK = NF // FB


def kernel(inputs, categorical_embeddings, continuous_embeddings, bias):
    mesh = plsc.VectorSubcoreMesh(
        core_axis_name="c", subcore_axis_name="s", num_cores=NC, num_subcores=NS
    )

    @functools.partial(
        pl.kernel,
        out_type=jax.ShapeDtypeStruct((NF * 2 * NBH * 8 * BH,), jnp.float32),
        mesh=mesh,
        compiler_params=pltpu.CompilerParams(
            needs_layout_passes=False, use_tc_tiling_on_sc=False
        ),
        scratch_types=[
            pltpu.VMEM((NF, BH), jnp.float32),
            pltpu.VMEM((NCAT, BH), jnp.int32),
            pltpu.VMEM((2, FB * BH, D), jnp.float32),
            pltpu.VMEM((2, FB * D * BH), jnp.float32),
            pltpu.VMEM((NF, D), jnp.float32),
            pltpu.VMEM((NCONT * D, D), jnp.float32),
            pltpu.VMEM((FB, L), jnp.int32),
            pltpu.VMEM((L + NCONT * D,), jnp.float32),
            pltpu.VMEM((L + NF,), jnp.float32),
            pltpu.SemaphoreType.DMA,
            pltpu.SemaphoreType.DMA,
            pltpu.SemaphoreType.DMA,
        ],
    )
    def sc_kernel(xt_hbm, table_hbm, cont_hbm, bias_hbm, out_hbm,
                  xchunk, idxb, gbuf, stag, bb, cesp, sbase, ce, bv,
                  gsem, osem0, osem1):
        osems = (osem0, osem1)
        wid = lax.axis_index("s") * NC + lax.axis_index("c")

        pltpu.sync_copy(cont_hbm, ce.at[pl.ds(L, NCONT * D)])
        pltpu.sync_copy(bias_hbm, bv.at[pl.ds(L, NF)])
        for f in range(NF):
            bb[f, :] = plsc.load_gather(bv, [jnp.full((L,), L + f, jnp.int32)])
        for f in range(NCONT):
            for d in range(D):
                cesp[f * D + d, :] = plsc.load_gather(
                    ce, [jnp.full((L,), L + f * D + d, jnp.int32)]
                )
        for fl in range(FB):
            sbase[fl, :] = (fl * D + lax.iota(jnp.int32, L)) * BH

        def fire_gathers(blk):
            p = blk & 1
            for fl in range(FB):
                fc = blk * FB - NCONT + fl
                pltpu.async_copy(
                    table_hbm.at[idxb.at[fc]],
                    gbuf.at[p, pl.ds(fl * BH, BH)], gsem,
                )

        def wait_gathers(blk):
            p = blk & 1
            for fl in range(FB):
                fc = blk * FB - NCONT + fl
                pltpu.make_async_copy(
                    table_hbm.at[idxb.at[fc]],
                    gbuf.at[p, pl.ds(fl * BH, BH)], gsem,
                ).wait()

        def drain_out(p):
            for _ in range(2 * FB):
                pltpu.make_async_copy(
                    stag.at[0, pl.ds(0, 8 * BH)],
                    out_hbm.at[pl.ds(0, 8 * BH)], osems[p],
                ).wait()

        def fire_out(blk, bh):
            p = blk & 1
            for fl in range(FB):
                fo = blk * FB + fl
                for dh in range(2):
                    r = ((fo * 2 + dh) * NBH + bh) * 8 * BH
                    pltpu.async_copy(
                        stag.at[p, pl.ds((fl * D + dh * 8) * BH, 8 * BH)],
                        out_hbm.at[pl.ds(r, 8 * BH)], osems[p],
                    )

        def tile_body(c, carry):
            bh = wid * TILES_PER_W + c
            b0 = bh * BH
            pltpu.sync_copy(xt_hbm.at[:, pl.ds(b0, BH)], xchunk)

            @plsc.parallel_loop(0, NCAT, unroll=1)
            def idx_body(f):
                offs = jnp.full((L,), 1, jnp.int32) + f * 10000
                for k in range(BH // L):
                    xv = xchunk[NCONT + f, pl.ds(L * k, L)]
                    idxb[f, pl.ds(L * k, L)] = xv.astype(jnp.int32) + offs

            fire_gathers(2)
            for blk in range(NBLK):
                p = blk & 1
                if 3 <= blk + 1 < NBLK:
                    fire_gathers(blk + 1)
                if blk >= 2:
                    wait_gathers(blk)
                if blk >= 2:
                    drain_out(p)
                if blk < 2:
                    def cont_body(fl, _):
                        f = blk * FB + fl
                        biasv = bb[f, :]
                        for k in range(BH // L):
                            xv = xchunk[f, pl.ds(L * k, L)]
                            for d in range(D):
                                ev = cesp[f * D + d, :]
                                stag[p, pl.ds((fl * D + d) * BH + L * k, L)] = (
                                    xv * ev + biasv
                                )
                        return _
                    lax.fori_loop(0, FB, cont_body, 0)
                else:
                    biases = [bb[blk * FB + fl, :] for fl in range(FB)]
                    sbs = [sbase[fl, :] for fl in range(FB)]
                    @plsc.parallel_loop(0, BH, unroll=1)
                    def sc_body(b):
                        bvv = jnp.full((L,), b, jnp.int32)
                        for fl in range(FB):
                            row = gbuf[p, fl * BH + b, :] + biases[fl]
                            plsc.store_scatter(
                                stag.at[p], [sbs[fl] + bvv], row
                            )
                fire_out(blk, bh)
            drain_out(0)
            drain_out(1)
            return carry

        lax.fori_loop(0, TILES_PER_W, tile_body, 0)

    out = sc_kernel(
        inputs.T, categorical_embeddings, continuous_embeddings.reshape(-1), bias
    )
    out = out.reshape(NF, 2, NBH, 8, BH).transpose(2, 4, 0, 1, 3)
    return out.reshape(BATCH, NF, D)
